# baseline (speedup 1.0000x reference)
import contextlib

import jax
import jax.numpy as jnp
from jax import lax
from jax.experimental import pallas as pl
from jax.experimental.pallas import tpu as pltpu
from jax.experimental.pallas import tpu_sc as plsc

N_NODES = 10000
N_EDGES = 320000
NODE_FEAT = 128
INPUT_ENC = 64

NT = 16
L = 16
NP = 10240
NSL = NP // NT
EPT = 20480
EPT_LAST = N_EDGES - 15 * EPT
UE = 32
MPNN_STEPS = 3


def _sc_body(ei_hbm, w_hbm,
             sd_v, pk_v, ufull_v, acc_v, red_v,
             wsl_v, cinv_v, usl_v, stage_sh, ush_sh, dma_sem):
    tid = lax.axis_index("s")
    nbase = tid * NSL
    zeros16 = jnp.zeros((L,), jnp.float32)
    ones16 = jnp.ones((L,), jnp.float32)
    n_chunks = jnp.where(tid < NT - 1, EPT // L, EPT_LAST // L)

    def load_edges():
        @pl.when(tid < NT - 1)
        def _():
            pltpu.async_copy(ei_hbm.at[:, pl.ds(tid * EPT, EPT)], sd_v,
                             dma_sem)

        @pl.when(tid == NT - 1)
        def _():
            pltpu.async_copy(ei_hbm.at[:, pl.ds((NT - 1) * EPT, EPT_LAST)],
                             sd_v.at[:, pl.ds(0, EPT_LAST)], dma_sem)

    def drain_edges():
        @pl.when(tid < NT - 1)
        def _():
            pltpu.make_async_copy(ei_hbm.at[:, pl.ds(tid * EPT, EPT)], sd_v,
                                  dma_sem).wait()

        @pl.when(tid == NT - 1)
        def _():
            pltpu.make_async_copy(
                ei_hbm.at[:, pl.ds((NT - 1) * EPT, EPT_LAST)],
                sd_v.at[:, pl.ds(0, EPT_LAST)], dma_sem).wait()

    load_edges()

    def zero_acc():
        def body(i, _):
            for k in range(8):
                acc_v[pl.ds(i * (8 * L) + k * L, L)] = zeros16
            return 0
        lax.fori_loop(0, NP // (8 * L), body, 0)

    def reduce_chunk(j):
        s = red_v[0, pl.ds(j * L, L)]
        for r in range(1, NT):
            s = s + red_v[r, pl.ds(j * L, L)]
        return s

    with contextlib.nullcontext():
        zero_acc()
        drain_edges()

        @plsc.parallel_loop(0, n_chunks, step=1, unroll=UE)
        def _(i):
            d = sd_v[1, pl.ds(i * L, L)]
            s = sd_v[0, pl.ds(i * L, L)]
            plsc.addupdate_scatter(acc_v, [d], ones16)
            pk_v[pl.ds(i * L, L)] = s | (d << 16)

        pltpu.sync_copy(acc_v, stage_sh.at[tid])
        plsc.subcore_barrier()
        pltpu.sync_copy(stage_sh.at[:, pl.ds(nbase, NSL)], red_v)

        @plsc.parallel_loop(0, NSL // L, step=1, unroll=4)
        def _(j):
            c = jnp.maximum(reduce_chunk(j), 1.0)
            cinv = 1.0 / c
            cinv_v[pl.ds(j * L, L)] = cinv
            gidx = nbase + j * L + lax.iota(jnp.int32, L)
            w = jnp.where(gidx < N_NODES, jnp.float32(1.0 / N_NODES), 0.0)
            wsl_v[pl.ds(j * L, L)] = w
            usl_v[pl.ds(j * L, L)] = w * cinv

        pltpu.sync_copy(usl_v, ush_sh.at[pl.ds(nbase, NSL)])
        plsc.subcore_barrier()

    for step in range(MPNN_STEPS):
        with contextlib.nullcontext():
            pltpu.async_copy(ush_sh, ufull_v, dma_sem)
            zero_acc()
            pltpu.make_async_copy(ush_sh, ufull_v, dma_sem).wait()

        with contextlib.nullcontext():
            @plsc.parallel_loop(0, n_chunks, step=1, unroll=UE)
            def _(i):
                pk = pk_v[pl.ds(i * L, L)]
                d = lax.shift_right_logical(pk, 16)
                s = pk & 0xFFFF
                vals = plsc.load_gather(ufull_v, [d])
                plsc.addupdate_scatter(acc_v, [s], vals)

        with contextlib.nullcontext():
            pltpu.sync_copy(acc_v, stage_sh.at[tid])
            plsc.subcore_barrier()
            pltpu.sync_copy(stage_sh.at[:, pl.ds(nbase, NSL)], red_v)

            @plsc.parallel_loop(0, NSL // L, step=1, unroll=4)
            def _(j):
                w = (wsl_v[pl.ds(j * L, L)] + reduce_chunk(j)) * 0.5
                wsl_v[pl.ds(j * L, L)] = w
                if step < MPNN_STEPS - 1:
                    usl_v[pl.ds(j * L, L)] = w * cinv_v[pl.ds(j * L, L)]

            if step < MPNN_STEPS - 1:
                pltpu.sync_copy(usl_v, ush_sh.at[pl.ds(nbase, NSL)])
                plsc.subcore_barrier()
            else:
                pltpu.sync_copy(wsl_v, w_hbm.at[pl.ds(nbase, NSL)])


def _sc_propagate(ei32):
    mesh = plsc.VectorSubcoreMesh(core_axis_name="c", subcore_axis_name="s",
                                  num_cores=1)
    kern = pl.kernel(
        _sc_body,
        out_type=jax.ShapeDtypeStruct((NP,), jnp.float32),
        mesh=mesh,
        compiler_params=pltpu.CompilerParams(needs_layout_passes=False),
        scratch_types=[
            pltpu.VMEM((2, EPT), jnp.int32),
            pltpu.VMEM((EPT,), jnp.int32),
            pltpu.VMEM((NP,), jnp.float32),
            pltpu.VMEM((NP,), jnp.float32),
            pltpu.VMEM((NT, NSL), jnp.float32),
            pltpu.VMEM((NSL,), jnp.float32),
            pltpu.VMEM((NSL,), jnp.float32),
            pltpu.VMEM((NSL,), jnp.float32),
            pltpu.VMEM_SHARED((NT, NP), jnp.float32),
            pltpu.VMEM_SHARED((NP,), jnp.float32),
            pltpu.SemaphoreType.DMA,
        ],
    )
    return kern(ei32)


def _enc_body(x_ref, win_ref, b_ref, h_ref):
    h = lax.dot_general(win_ref[...], x_ref[...],
                        (((1,), (1,)), ((), ())),
                        preferred_element_type=jnp.float32)
    h_ref[...] = jnp.maximum(h + b_ref[...], 0.0).astype(jnp.bfloat16)


def _fin_body(w_ref, h_ref, wp_ref, bp_ref, out_ref):
    wv = w_ref[...][:, :N_NODES]
    h = h_ref[...].astype(jnp.float32)
    s = jnp.sum(h * wv, axis=1, keepdims=True)
    out_ref[...] = jnp.sum(s * wp_ref[...], axis=0, keepdims=True) + bp_ref[...]


def kernel(x, edge_index, W_in, b_in, W_pred, b_pred):
    h0t = pl.pallas_call(
        _enc_body,
        out_shape=jax.ShapeDtypeStruct((INPUT_ENC, N_NODES), jnp.bfloat16),
    )(x, W_in, b_in.reshape(INPUT_ENC, 1))

    w3 = _sc_propagate(edge_index.astype(jnp.int32))

    out = pl.pallas_call(
        _fin_body,
        out_shape=jax.ShapeDtypeStruct((1, 1), jnp.float32),
    )(w3.reshape(1, NP), h0t, W_pred[:, :INPUT_ENC].reshape(INPUT_ENC, 1),
      b_pred.reshape(1, 1))
    return out.reshape(1)


# speedup vs baseline: 1.0193x; 1.0193x over previous
import contextlib

import jax
import jax.numpy as jnp
from jax import lax
from jax.experimental import pallas as pl
from jax.experimental.pallas import tpu as pltpu
from jax.experimental.pallas import tpu_sc as plsc

N_NODES = 10000
N_EDGES = 320000
NODE_FEAT = 128
INPUT_ENC = 64

NT = 16
L = 16
NP = 10240
NSL = NP // NT
EPT = 20480
EPT_LAST = N_EDGES - 15 * EPT
UE = 16
MPNN_STEPS = 3


def _sc_body(ei_hbm, w_hbm,
             sd_v, pk_v, ufull_v, acc_v, red_v,
             wsl_v, cinv_v, usl_v, stage_sh, ush_sh, dma_sem):
    tid = lax.axis_index("s")
    nbase = tid * NSL
    zeros16 = jnp.zeros((L,), jnp.float32)
    ones16 = jnp.ones((L,), jnp.float32)
    n_chunks = jnp.where(tid < NT - 1, EPT // L, EPT_LAST // L)

    def load_edges():
        @pl.when(tid < NT - 1)
        def _():
            pltpu.async_copy(ei_hbm.at[:, pl.ds(tid * EPT, EPT)], sd_v,
                             dma_sem)

        @pl.when(tid == NT - 1)
        def _():
            pltpu.async_copy(ei_hbm.at[:, pl.ds((NT - 1) * EPT, EPT_LAST)],
                             sd_v.at[:, pl.ds(0, EPT_LAST)], dma_sem)

    def drain_edges():
        @pl.when(tid < NT - 1)
        def _():
            pltpu.make_async_copy(ei_hbm.at[:, pl.ds(tid * EPT, EPT)], sd_v,
                                  dma_sem).wait()

        @pl.when(tid == NT - 1)
        def _():
            pltpu.make_async_copy(
                ei_hbm.at[:, pl.ds((NT - 1) * EPT, EPT_LAST)],
                sd_v.at[:, pl.ds(0, EPT_LAST)], dma_sem).wait()

    load_edges()

    def zero_acc():
        def body(i, _):
            for k in range(8):
                acc_v[pl.ds(i * (8 * L) + k * L, L)] = zeros16
            return 0
        lax.fori_loop(0, NP // (8 * L), body, 0)

    def reduce_chunk(j):
        s = red_v[0, pl.ds(j * L, L)]
        for r in range(1, NT):
            s = s + red_v[r, pl.ds(j * L, L)]
        return s

    with contextlib.nullcontext():
        zero_acc()
        drain_edges()

        @plsc.parallel_loop(0, n_chunks, step=1, unroll=UE)
        def _(i):
            d = sd_v[1, pl.ds(i * L, L)]
            s = sd_v[0, pl.ds(i * L, L)]
            plsc.addupdate_scatter(acc_v, [d], ones16)
            pk_v[pl.ds(i * L, L)] = s | (d << 16)

        pltpu.sync_copy(acc_v, stage_sh.at[tid])
        plsc.subcore_barrier()
        pltpu.sync_copy(stage_sh.at[:, pl.ds(nbase, NSL)], red_v)

        @plsc.parallel_loop(0, NSL // L, step=1, unroll=4)
        def _(j):
            c = jnp.maximum(reduce_chunk(j), 1.0)
            cinv = 1.0 / c
            cinv_v[pl.ds(j * L, L)] = cinv
            gidx = nbase + j * L + lax.iota(jnp.int32, L)
            w = jnp.where(gidx < N_NODES, jnp.float32(1.0 / N_NODES), 0.0)
            wsl_v[pl.ds(j * L, L)] = w
            usl_v[pl.ds(j * L, L)] = w * cinv

        pltpu.sync_copy(usl_v, ush_sh.at[pl.ds(nbase, NSL)])
        plsc.subcore_barrier()

    for step in range(MPNN_STEPS):
        with contextlib.nullcontext():
            pltpu.async_copy(ush_sh, ufull_v, dma_sem)
            zero_acc()
            pltpu.make_async_copy(ush_sh, ufull_v, dma_sem).wait()

        with contextlib.nullcontext():
            @plsc.parallel_loop(0, n_chunks, step=1, unroll=UE)
            def _(i):
                pk = pk_v[pl.ds(i * L, L)]
                d = lax.shift_right_logical(pk, 16)
                s = pk & 0xFFFF
                vals = plsc.load_gather(ufull_v, [d])
                plsc.addupdate_scatter(acc_v, [s], vals)

        with contextlib.nullcontext():
            pltpu.sync_copy(acc_v, stage_sh.at[tid])
            plsc.subcore_barrier()
            pltpu.sync_copy(stage_sh.at[:, pl.ds(nbase, NSL)], red_v)

            @plsc.parallel_loop(0, NSL // L, step=1, unroll=4)
            def _(j):
                w = (wsl_v[pl.ds(j * L, L)] + reduce_chunk(j)) * 0.5
                wsl_v[pl.ds(j * L, L)] = w
                if step < MPNN_STEPS - 1:
                    usl_v[pl.ds(j * L, L)] = w * cinv_v[pl.ds(j * L, L)]

            if step < MPNN_STEPS - 1:
                pltpu.sync_copy(usl_v, ush_sh.at[pl.ds(nbase, NSL)])
                plsc.subcore_barrier()
            else:
                pltpu.sync_copy(wsl_v, w_hbm.at[pl.ds(nbase, NSL)])


def _sc_propagate(ei32):
    mesh = plsc.VectorSubcoreMesh(core_axis_name="c", subcore_axis_name="s",
                                  num_cores=1)
    kern = pl.kernel(
        _sc_body,
        out_type=jax.ShapeDtypeStruct((NP,), jnp.float32),
        mesh=mesh,
        compiler_params=pltpu.CompilerParams(needs_layout_passes=False),
        scratch_types=[
            pltpu.VMEM((2, EPT), jnp.int32),
            pltpu.VMEM((EPT,), jnp.int32),
            pltpu.VMEM((NP,), jnp.float32),
            pltpu.VMEM((NP,), jnp.float32),
            pltpu.VMEM((NT, NSL), jnp.float32),
            pltpu.VMEM((NSL,), jnp.float32),
            pltpu.VMEM((NSL,), jnp.float32),
            pltpu.VMEM((NSL,), jnp.float32),
            pltpu.VMEM_SHARED((NT, NP), jnp.float32),
            pltpu.VMEM_SHARED((NP,), jnp.float32),
            pltpu.SemaphoreType.DMA,
        ],
    )
    return kern(ei32)


def _enc_body(x_ref, win_ref, b_ref, h_ref):
    h = lax.dot_general(win_ref[...], x_ref[...],
                        (((1,), (1,)), ((), ())),
                        preferred_element_type=jnp.float32)
    h_ref[...] = jnp.maximum(h + b_ref[...], 0.0).astype(jnp.bfloat16)


def _fin_body(w_ref, h_ref, wp_ref, bp_ref, out_ref):
    wv = w_ref[...][:, :N_NODES]
    h = h_ref[...].astype(jnp.float32)
    s = jnp.sum(h * wv, axis=1, keepdims=True)
    out_ref[...] = jnp.sum(s * wp_ref[...], axis=0, keepdims=True) + bp_ref[...]


def kernel(x, edge_index, W_in, b_in, W_pred, b_pred):
    h0t = pl.pallas_call(
        _enc_body,
        out_shape=jax.ShapeDtypeStruct((INPUT_ENC, N_NODES), jnp.bfloat16),
    )(x, W_in, b_in.reshape(INPUT_ENC, 1))

    w3 = _sc_propagate(edge_index.astype(jnp.int32))

    out = pl.pallas_call(
        _fin_body,
        out_shape=jax.ShapeDtypeStruct((1, 1), jnp.float32),
    )(w3.reshape(1, NP), h0t, W_pred[:, :INPUT_ENC].reshape(INPUT_ENC, 1),
      b_pred.reshape(1, 1))
    return out.reshape(1)
